# baseline (speedup 1.0000x reference)
import functools

import jax
import jax.numpy as jnp
from jax import lax
from jax.experimental import pallas as pl
from jax.experimental.pallas import tpu as pltpu
from jax.experimental.pallas import tpu_sc as plsc

_MT = 512
_KT = 2048
_D0 = 256
_DAUG = 264
_LOSS_SCALE = 1.25

_NC, _NS = 2, 16
_CH = 128


def _dist_argmin_body(z_ref, cb_ref, idx_ref, loss_ref, run_d, run_a):
    i = pl.program_id(0)
    k = pl.program_id(1)
    nk = pl.num_programs(1)
    zt = z_ref[...]
    ct = cb_ref[...]
    zo = zt[:, :_D0]
    a = 0.25 * jnp.sum(zo * zo, axis=1, keepdims=True)
    m = lax.dot_general(zt, ct, (((1,), (1,)), ((), ())),
                        preferred_element_type=jnp.float32)
    d = jnp.maximum(m, -a)
    lmin = jnp.min(d, axis=1, keepdims=True)
    kio = lax.broadcasted_iota(jnp.int32, d.shape, 1)
    larg = jnp.min(jnp.where(d == lmin, kio, jnp.int32(d.shape[1])),
                   axis=1, keepdims=True) + k * d.shape[1]

    @pl.when(k == 0)
    def _():
        run_d[...] = lmin
        run_a[...] = larg

    @pl.when(k > 0)
    def _():
        better = lmin < run_d[...]
        run_d[...] = jnp.where(better, lmin, run_d[...])
        run_a[...] = jnp.where(better, larg, run_a[...])

    @pl.when(k == nk - 1)
    def _():
        idx_ref[...] = run_a[...]
        v = jnp.sum(run_d[...] + a)
        loss_ref[0, 0] = jnp.where(i == 0, v, loss_ref[0, 0] + v)


def _cb_norm_body(cb_ref, out_ref):
    ct = cb_ref[...]
    out_ref[...] = jnp.sum(ct * ct, axis=1, keepdims=True)


def _cb_norms(codebook):
    k, d = codebook.shape
    bk = 1024
    return pl.pallas_call(
        _cb_norm_body,
        grid=(k // bk,),
        in_specs=[pl.BlockSpec((bk, d), lambda i: (i, 0))],
        out_specs=pl.BlockSpec((bk, 1), lambda i: (i, 0)),
        out_shape=jax.ShapeDtypeStruct((k, 1), jnp.float32),
    )(codebook)


def _distance_argmin(z_aug, cb_aug):
    m, d = z_aug.shape
    k = cb_aug.shape[0]
    return pl.pallas_call(
        _dist_argmin_body,
        grid=(m // _MT, k // _KT),
        in_specs=[
            pl.BlockSpec((_MT, d), lambda i, j: (i, 0)),
            pl.BlockSpec((_KT, d), lambda i, j: (j, 0)),
        ],
        out_specs=[
            pl.BlockSpec((_MT, 1), lambda i, j: (i, 0)),
            pl.BlockSpec(memory_space=pltpu.SMEM),
        ],
        out_shape=[
            jax.ShapeDtypeStruct((m, 1), jnp.int32),
            jax.ShapeDtypeStruct((1, 1), jnp.float32),
        ],
        scratch_shapes=[
            pltpu.VMEM((_MT, 1), jnp.float32),
            pltpu.VMEM((_MT, 1), jnp.int32),
        ],
    )(z_aug, cb_aug)


def _gather_rows(codebook, idx_flat):
    m = idx_flat.shape[0]
    d = codebook.shape[1]
    nw = _NC * _NS
    per_w = m // nw
    nch = per_w // _CH
    mesh = plsc.VectorSubcoreMesh(core_axis_name="c", subcore_axis_name="s")

    @functools.partial(
        pl.kernel,
        out_type=jax.ShapeDtypeStruct((m, d), jnp.float32),
        mesh=mesh,
        scratch_types=[
            pltpu.VMEM((_CH,), jnp.int32),
            pltpu.VMEM((_CH,), jnp.int32),
            pltpu.VMEM((_CH, d), jnp.float32),
            pltpu.VMEM((_CH, d), jnp.float32),
            pltpu.SemaphoreType.DMA,
            pltpu.SemaphoreType.DMA,
        ],
    )
    def gk(cb_hbm, idx_hbm, out_hbm, idx0, idx1, rows0, rows1, sem0, sem1):
        wid = lax.axis_index("s") * _NC + lax.axis_index("c")
        base0 = wid * per_w
        idx = (idx0, idx1)
        rows = (rows0, rows1)
        sems = (sem0, sem1)
        pltpu.sync_copy(idx_hbm.at[pl.ds(base0, _CH)], idx0)
        handles = [pltpu.async_copy(cb_hbm.at[idx0], rows0, sem0)]
        for c in range(nch):
            cur = c % 2
            if c + 1 < nch:
                nxt = (c + 1) % 2
                pltpu.sync_copy(
                    idx_hbm.at[pl.ds(base0 + (c + 1) * _CH, _CH)], idx[nxt])
                handles.append(
                    pltpu.async_copy(cb_hbm.at[idx[nxt]], rows[nxt], sems[nxt]))
            handles[c].wait()
            pltpu.sync_copy(rows[cur], out_hbm.at[pl.ds(base0 + c * _CH, _CH)])

    return gk(codebook, idx_flat)


def kernel(z_e, codebook):
    b, l, d = z_e.shape
    m = b * l
    k = codebook.shape[0]
    z_flat = z_e.reshape(m, d)
    cnorm = _cb_norms(codebook)
    pad = _DAUG - d - 1
    z_aug = jnp.concatenate(
        [-2.0 * z_flat,
         jnp.ones((m, 1), jnp.float32),
         jnp.zeros((m, pad), jnp.float32)], axis=1)
    cb_aug = jnp.concatenate(
        [codebook, cnorm, jnp.zeros((k, pad), jnp.float32)], axis=1)
    idx2d, loss_sum = _distance_argmin(z_aug, cb_aug)
    idx_flat = idx2d.reshape(m)
    z_q = _gather_rows(codebook, idx_flat).reshape(b, l, d)
    vq_loss = (_LOSS_SCALE * loss_sum[0, 0]) / (m * d)
    return (z_q, vq_loss, idx_flat.reshape(b, l))


# speedup vs baseline: 1.2172x; 1.2172x over previous
import functools

import jax
import jax.numpy as jnp
from jax import lax
from jax.experimental import pallas as pl
from jax.experimental.pallas import tpu as pltpu
from jax.experimental.pallas import tpu_sc as plsc

_MT = 512
_KT = 2048
_LOSS_SCALE = 1.25

_NC, _NS = 2, 16
_CH = 128


def _dist_argmin_body(z_ref, cb_ref, idx_ref, loss_ref, run_d, run_a):
    i = pl.program_id(0)
    k = pl.program_id(1)
    nk = pl.num_programs(1)
    zt = z_ref[...]
    ct = cb_ref[...]
    a = jnp.sum(zt * zt, axis=1, keepdims=True)
    c2 = jnp.sum(ct * ct, axis=1)[None, :]
    m = lax.dot_general(zt, ct, (((1,), (1,)), ((), ())),
                        preferred_element_type=jnp.float32)
    d = jnp.maximum(a - 2.0 * m + c2, 0.0)
    lmin = jnp.min(d, axis=1, keepdims=True)
    kio = lax.broadcasted_iota(jnp.int32, d.shape, 1)
    larg = jnp.min(jnp.where(d == lmin, kio, jnp.int32(d.shape[1])),
                   axis=1, keepdims=True) + k * d.shape[1]

    @pl.when(k == 0)
    def _():
        run_d[...] = lmin
        run_a[...] = larg

    @pl.when(k > 0)
    def _():
        better = lmin < run_d[...]
        run_d[...] = jnp.where(better, lmin, run_d[...])
        run_a[...] = jnp.where(better, larg, run_a[...])

    @pl.when(k == nk - 1)
    def _():
        idx_ref[...] = run_a[...]
        v = jnp.sum(run_d[...])
        loss_ref[0, 0] = jnp.where(i == 0, v, loss_ref[0, 0] + v)


def _distance_argmin(z_flat, codebook):
    m, d = z_flat.shape
    k = codebook.shape[0]
    return pl.pallas_call(
        _dist_argmin_body,
        grid=(m // _MT, k // _KT),
        in_specs=[
            pl.BlockSpec((_MT, d), lambda i, j: (i, 0)),
            pl.BlockSpec((_KT, d), lambda i, j: (j, 0)),
        ],
        out_specs=[
            pl.BlockSpec((_MT, 1), lambda i, j: (i, 0)),
            pl.BlockSpec(memory_space=pltpu.SMEM),
        ],
        out_shape=[
            jax.ShapeDtypeStruct((m, 1), jnp.int32),
            jax.ShapeDtypeStruct((1, 1), jnp.float32),
        ],
        scratch_shapes=[
            pltpu.VMEM((_MT, 1), jnp.float32),
            pltpu.VMEM((_MT, 1), jnp.int32),
        ],
    )(z_flat, codebook)


def _gather_rows(codebook, idx_flat):
    m = idx_flat.shape[0]
    d = codebook.shape[1]
    nw = _NC * _NS
    per_w = m // nw
    nch = per_w // _CH
    mesh = plsc.VectorSubcoreMesh(core_axis_name="c", subcore_axis_name="s")

    @functools.partial(
        pl.kernel,
        out_type=jax.ShapeDtypeStruct((m, d), jnp.float32),
        mesh=mesh,
        scratch_types=[
            pltpu.VMEM((_CH,), jnp.int32),
            pltpu.VMEM((_CH,), jnp.int32),
            pltpu.VMEM((_CH, d), jnp.float32),
            pltpu.VMEM((_CH, d), jnp.float32),
            pltpu.SemaphoreType.DMA,
            pltpu.SemaphoreType.DMA,
        ],
    )
    def gk(cb_hbm, idx_hbm, out_hbm, idx0, idx1, rows0, rows1, sem0, sem1):
        wid = lax.axis_index("s") * _NC + lax.axis_index("c")
        base0 = wid * per_w
        idx = (idx0, idx1)
        rows = (rows0, rows1)
        sems = (sem0, sem1)
        pltpu.sync_copy(idx_hbm.at[pl.ds(base0, _CH)], idx0)
        handles = [pltpu.async_copy(cb_hbm.at[idx0], rows0, sem0)]
        for c in range(nch):
            cur = c % 2
            if c + 1 < nch:
                nxt = (c + 1) % 2
                pltpu.sync_copy(
                    idx_hbm.at[pl.ds(base0 + (c + 1) * _CH, _CH)], idx[nxt])
                handles.append(
                    pltpu.async_copy(cb_hbm.at[idx[nxt]], rows[nxt], sems[nxt]))
            handles[c].wait()
            pltpu.sync_copy(rows[cur], out_hbm.at[pl.ds(base0 + c * _CH, _CH)])

    return gk(codebook, idx_flat)


def kernel(z_e, codebook):
    b, l, d = z_e.shape
    z_flat = z_e.reshape(b * l, d)
    idx2d, loss_sum = _distance_argmin(z_flat, codebook)
    idx_flat = idx2d.reshape(b * l)
    z_q = _gather_rows(codebook, idx_flat).reshape(b, l, d)
    vq_loss = (_LOSS_SCALE * loss_sum[0, 0]) / (b * l * d)
    return (z_q, vq_loss, idx_flat.reshape(b, l))
